# baseline (speedup 1.0000x reference)
import functools

import jax
import jax.numpy as jnp
from jax import lax
from jax.experimental import pallas as pl
from jax.experimental.pallas import tpu as pltpu
from jax.experimental.pallas import tpu_sc as plsc

_KC = 256


def _fused_body(a_ref, x_ref, k_ref, kv_ref, ki_ref, acc_ref, s_full, rank_ref):
    n = a_ref.shape[2]
    blk = a_ref.shape[1]
    i = pl.program_id(1)
    acc_ref[...] = jnp.dot(a_ref[0, :, 0:_KC], x_ref[0, 0:_KC, :],
                           preferred_element_type=jnp.float32)
    for kc in range(1, n // _KC):
        acc_ref[...] = acc_ref[...] + jnp.dot(
            a_ref[0, :, kc * _KC:(kc + 1) * _KC],
            x_ref[0, kc * _KC:(kc + 1) * _KC, :],
            preferred_element_type=jnp.float32)
    s_full[pl.ds(i * blk, blk), :] = jnp.dot(
        acc_ref[...], k_ref[...], preferred_element_type=jnp.float32)

    @pl.when(i == n // blk - 1)
    def _():
        k = n // 2
        ic_sz = 512
        s_col = s_full[...]
        s_row = jnp.transpose(s_col)
        m = jnp.max(s_row)
        u_row = jnp.exp(s_row - m)
        c = jnp.sum(u_row)
        q_row = u_row / c
        iota_row = lax.broadcasted_iota(jnp.int32, (1, n), 1)
        q_col = jnp.exp(s_col - m) / c

        for ci in range(n // ic_sz):
            qc = q_col[ci * ic_sz:(ci + 1) * ic_sz, :]
            icol = lax.broadcasted_iota(jnp.int32, (ic_sz, 1), 0) + ci * ic_sz
            gt = (q_row > qc)
            eqlow = (q_row == qc) & (iota_row < icol)
            rank = jnp.sum(gt.astype(jnp.int32) + eqlow.astype(jnp.int32),
                           axis=1, keepdims=True)
            rank_ref[ci * ic_sz:(ci + 1) * ic_sz, :] = rank

        p_row = lax.broadcasted_iota(jnp.int32, (1, k), 1)
        kv_acc = jnp.zeros((1, k), jnp.float32)
        ki_acc = jnp.zeros((1, k), jnp.int32)
        for ci in range(n // ic_sz):
            qc = q_col[ci * ic_sz:(ci + 1) * ic_sz, :]
            icol = lax.broadcasted_iota(jnp.int32, (ic_sz, 1), 0) + ci * ic_sz
            rank_c = rank_ref[ci * ic_sz:(ci + 1) * ic_sz, :]
            hit = (rank_c == p_row)
            kv_acc = kv_acc + jnp.sum(jnp.where(hit, qc, 0.0),
                                      axis=0, keepdims=True)
            ki_acc = ki_acc + jnp.sum(jnp.where(hit, icol, 0),
                                      axis=0, keepdims=True)
        kv_ref[0] = kv_acc
        ki_ref[0] = ki_acc


_NC, _NS = 2, 16
_NW = _NC * _NS
_CH = 8


def _make_sc_gather(B, N, FP, K):
    rows_pw = K // _NW
    mesh = plsc.VectorSubcoreMesh(core_axis_name="c", subcore_axis_name="s")

    @functools.partial(
        pl.kernel, mesh=mesh,
        compiler_params=pltpu.CompilerParams(needs_layout_passes=False),
        out_type=[jax.ShapeDtypeStruct((B, K, K), jnp.float32),
                  jax.ShapeDtypeStruct((B, K, FP), jnp.float32)],
        scratch_types=[
            pltpu.VMEM((K,), jnp.int32),
            pltpu.VMEM((rows_pw,), jnp.int32),
            pltpu.VMEM((_CH, N), jnp.float32),
            pltpu.VMEM((_CH, N), jnp.float32),
            pltpu.VMEM((_CH, K), jnp.float32),
            pltpu.VMEM((_CH, K), jnp.float32),
            pltpu.VMEM((rows_pw, FP), jnp.float32),
            pltpu.SemaphoreType.DMA,
            pltpu.SemaphoreType.DMA,
            pltpu.SemaphoreType.DMA,
            pltpu.SemaphoreType.DMA,
            pltpu.SemaphoreType.DMA,
        ],
    )
    def sc_gather(ki_hbm, as_hbm, xs_hbm, aout_hbm, xout_hbm,
                  idx_all, idx_rows, rows_v0, rows_v1, out_v0, out_v1,
                  xrows_v, sem, sem0, sem1, semw0, semw1):
        wid = lax.axis_index("s") * _NC + lax.axis_index("c")
        base_p = wid * rows_pw
        n_chunks = rows_pw // _CH

        for b in range(B):
            pltpu.sync_copy(ki_hbm.at[b], idx_all)
            pltpu.sync_copy(ki_hbm.at[b, pl.ds(base_p, rows_pw)], idx_rows)
            pltpu.async_copy(xs_hbm.at[b].at[idx_rows], xrows_v, sem).wait()
            pltpu.sync_copy(xrows_v, xout_hbm.at[b, pl.ds(base_p, rows_pw)])

            def select_cols(rows_buf, c, ob, sw):
                def col_body(pp, carry3):
                    ivs = [idx_all[pl.ds(pp * 128 + q * 16, 16)]
                           for q in range(8)]
                    for q in range(8):
                        off = pp * 128 + q * 16
                        for i in range(_CH):
                            rv = jnp.full((16,), i, jnp.int32)
                            ob[i, pl.ds(off, 16)] = plsc.load_gather(
                                rows_buf, [rv, ivs[q]])
                    return carry3

                lax.fori_loop(0, K // 128, col_body, 0)
                pltpu.async_copy(
                    ob, aout_hbm.at[b, pl.ds(base_p + c * _CH, _CH)], sw)

            def chunk_dma(c, buf, s):
                idxs = idx_rows.at[pl.ds(c * _CH, _CH)]
                return pltpu.async_copy(as_hbm.at[b].at[idxs], buf, s)

            def wait_buf(buf, s):
                pltpu.make_async_copy(
                    as_hbm.at[b].at[idx_rows.at[pl.ds(0, _CH)]],
                    buf, s).wait()

            def wait_out(ob, sw):
                pltpu.make_async_copy(
                    ob, aout_hbm.at[b, pl.ds(base_p, _CH)], sw).wait()

            chunk_dma(0, rows_v0, sem0)

            def pair_body(g, carry2):
                c0 = 2 * g
                chunk_dma(c0 + 1, rows_v1, sem1)
                wait_buf(rows_v0, sem0)

                @pl.when(g > 0)
                def _():
                    wait_out(out_v0, semw0)

                select_cols(rows_v0, c0, out_v0, semw0)

                @pl.when(c0 + 2 < n_chunks)
                def _():
                    chunk_dma(c0 + 2, rows_v0, sem0)

                wait_buf(rows_v1, sem1)

                @pl.when(g > 0)
                def _():
                    wait_out(out_v1, semw1)

                select_cols(rows_v1, c0 + 1, out_v1, semw1)
                return carry2

            lax.fori_loop(0, n_chunks // 2, pair_body, 0)
            wait_out(out_v0, semw0)
            wait_out(out_v1, semw1)

    return sc_gather


def kernel(Xs, As, attn_kernel):
    B, N, F = Xs.shape
    K = N // 2
    BLK = 512
    FP = 128
    Xs_pad = jnp.pad(Xs, ((0, 0), (0, 0), (0, FP - F)))

    fused = pl.pallas_call(
        _fused_body,
        grid=(1, N // BLK),
        in_specs=[
            pl.BlockSpec((1, BLK, N), lambda b, i: (b, i, 0)),
            pl.BlockSpec((1, N, F), lambda b, i: (b, 0, 0)),
            pl.BlockSpec((F, 1), lambda b, i: (0, 0)),
        ],
        out_specs=[
            pl.BlockSpec((1, 1, K), lambda b, i: (b, 0, 0)),
            pl.BlockSpec((1, 1, K), lambda b, i: (b, 0, 0)),
        ],
        out_shape=[
            jax.ShapeDtypeStruct((1, 1, K), jnp.float32),
            jax.ShapeDtypeStruct((1, 1, K), jnp.int32),
        ],
        scratch_shapes=[
            pltpu.VMEM((BLK, F), jnp.float32),
            pltpu.VMEM((N, 1), jnp.float32),
            pltpu.VMEM((N, 1), jnp.int32),
        ],
    )
    sc_gather = _make_sc_gather(1, N, FP, K)

    kvs, aouts, xouts = [], [], []
    for b in range(B):
        kv_b, ki_b = fused(As[b:b + 1], Xs[b:b + 1], attn_kernel)
        ao_b, xo_b = sc_gather(jnp.reshape(ki_b, (1, K)),
                               As[b:b + 1], Xs_pad[b:b + 1])
        kvs.append(jnp.reshape(kv_b, (1, K)))
        aouts.append(ao_b)
        xouts.append(xo_b[:, :, :F])
    keep_values = jnp.concatenate(kvs, axis=0)
    As_out = jnp.concatenate(aouts, axis=0)
    Xs_out = jnp.concatenate(xouts, axis=0)
    return (Xs_out, As_out, keep_values)


# speedup vs baseline: 1.4540x; 1.4540x over previous
import functools

import jax
import jax.numpy as jnp
from jax import lax
from jax.experimental import pallas as pl
from jax.experimental.pallas import tpu as pltpu
from jax.experimental.pallas import tpu_sc as plsc

_KC = 256


def _fused_body(a_ref, x_ref, k_ref, kv_ref, ki_ref, acc_ref, s_full, rank_ref):
    n = a_ref.shape[2]
    blk = a_ref.shape[1]
    i = pl.program_id(1)
    acc_ref[...] = jnp.dot(a_ref[0, :, 0:_KC], x_ref[0, 0:_KC, :],
                           preferred_element_type=jnp.float32)
    for kc in range(1, n // _KC):
        acc_ref[...] = acc_ref[...] + jnp.dot(
            a_ref[0, :, kc * _KC:(kc + 1) * _KC],
            x_ref[0, kc * _KC:(kc + 1) * _KC, :],
            preferred_element_type=jnp.float32)
    s_full[pl.ds(i * blk, blk), :] = jnp.dot(
        acc_ref[...], k_ref[...], preferred_element_type=jnp.float32)

    @pl.when(i == n // blk - 1)
    def _():
        k = n // 2
        ic_sz = 512
        s_col = s_full[...]
        s_row = jnp.transpose(s_col)
        m = jnp.max(s_row)
        u_row = jnp.exp(s_row - m)
        c = jnp.sum(u_row)
        q_row = u_row / c
        iota_row = lax.broadcasted_iota(jnp.int32, (1, n), 1)
        q_col = jnp.exp(s_col - m) / c

        for ci in range(n // ic_sz):
            qc = q_col[ci * ic_sz:(ci + 1) * ic_sz, :]
            icol = lax.broadcasted_iota(jnp.int32, (ic_sz, 1), 0) + ci * ic_sz
            gt = (q_row > qc)
            eqlow = (q_row == qc) & (iota_row < icol)
            rank = jnp.sum(gt.astype(jnp.int32) + eqlow.astype(jnp.int32),
                           axis=1, keepdims=True)
            rank_ref[ci * ic_sz:(ci + 1) * ic_sz, :] = rank

        p_row = lax.broadcasted_iota(jnp.int32, (1, k), 1)
        kv_acc = jnp.zeros((1, k), jnp.float32)
        ki_acc = jnp.zeros((1, k), jnp.int32)
        for ci in range(n // ic_sz):
            qc = q_col[ci * ic_sz:(ci + 1) * ic_sz, :]
            icol = lax.broadcasted_iota(jnp.int32, (ic_sz, 1), 0) + ci * ic_sz
            rank_c = rank_ref[ci * ic_sz:(ci + 1) * ic_sz, :]
            hit = (rank_c == p_row)
            kv_acc = kv_acc + jnp.sum(jnp.where(hit, qc, 0.0),
                                      axis=0, keepdims=True)
            ki_acc = ki_acc + jnp.sum(jnp.where(hit, icol, 0),
                                      axis=0, keepdims=True)
        kv_ref[0] = kv_acc
        ki_ref[0] = ki_acc


_NC, _NS = 2, 16
_NW = _NC * _NS
_CH = 8


def _make_sc_gather(B, N, FP, K):
    rows_pw = K // _NW
    mesh = plsc.VectorSubcoreMesh(core_axis_name="c", subcore_axis_name="s")

    @functools.partial(
        pl.kernel, mesh=mesh,
        compiler_params=pltpu.CompilerParams(needs_layout_passes=False),
        out_type=[jax.ShapeDtypeStruct((B, K, K), jnp.float32),
                  jax.ShapeDtypeStruct((B, K, FP), jnp.float32)],
        scratch_types=[
            pltpu.VMEM((K,), jnp.int32),
            pltpu.VMEM((rows_pw,), jnp.int32),
            pltpu.VMEM((_CH, N), jnp.float32),
            pltpu.VMEM((_CH, N), jnp.float32),
            pltpu.VMEM((_CH, K), jnp.float32),
            pltpu.VMEM((_CH, K), jnp.float32),
            pltpu.VMEM((rows_pw, FP), jnp.float32),
            pltpu.SemaphoreType.DMA,
            pltpu.SemaphoreType.DMA,
            pltpu.SemaphoreType.DMA,
            pltpu.SemaphoreType.DMA,
            pltpu.SemaphoreType.DMA,
        ],
    )
    def sc_gather(ki_hbm, as_hbm, xs_hbm, aout_hbm, xout_hbm,
                  idx_all, idx_rows, rows_v0, rows_v1, out_v0, out_v1,
                  xrows_v, sem, sem0, sem1, semw0, semw1):
        wid = lax.axis_index("s") * _NC + lax.axis_index("c")
        base_p = wid * rows_pw
        n_chunks = rows_pw // _CH

        for b in range(B):
            pltpu.sync_copy(ki_hbm.at[b], idx_all)
            pltpu.sync_copy(ki_hbm.at[b, pl.ds(base_p, rows_pw)], idx_rows)
            pltpu.async_copy(xs_hbm.at[b].at[idx_rows], xrows_v, sem).wait()
            pltpu.sync_copy(xrows_v, xout_hbm.at[b, pl.ds(base_p, rows_pw)])

            def select_cols(rows_buf, c, ob, sw):
                def col_body(pp, carry3):
                    ivs = [idx_all[pl.ds(pp * 128 + q * 16, 16)]
                           for q in range(8)]
                    for q in range(8):
                        off = pp * 128 + q * 16
                        for i in range(_CH):
                            rv = jnp.full((16,), i, jnp.int32)
                            ob[i, pl.ds(off, 16)] = plsc.load_gather(
                                rows_buf, [rv, ivs[q]])
                    return carry3

                lax.fori_loop(0, K // 128, col_body, 0)
                pltpu.async_copy(
                    ob, aout_hbm.at[b, pl.ds(base_p + c * _CH, _CH)], sw)

            def chunk_dma(c, buf, s):
                idxs = idx_rows.at[pl.ds(c * _CH, _CH)]
                return pltpu.async_copy(as_hbm.at[b].at[idxs], buf, s)

            def wait_buf(buf, s):
                pltpu.make_async_copy(
                    as_hbm.at[b].at[idx_rows.at[pl.ds(0, _CH)]],
                    buf, s).wait()

            def wait_out(ob, sw):
                pltpu.make_async_copy(
                    ob, aout_hbm.at[b, pl.ds(base_p, _CH)], sw).wait()

            chunk_dma(0, rows_v0, sem0)

            def pair_body(g, carry2):
                c0 = 2 * g
                chunk_dma(c0 + 1, rows_v1, sem1)
                wait_buf(rows_v0, sem0)

                @pl.when(g > 0)
                def _():
                    wait_out(out_v0, semw0)

                select_cols(rows_v0, c0, out_v0, semw0)

                @pl.when(c0 + 2 < n_chunks)
                def _():
                    chunk_dma(c0 + 2, rows_v0, sem0)

                wait_buf(rows_v1, sem1)

                @pl.when(g > 0)
                def _():
                    wait_out(out_v1, semw1)

                select_cols(rows_v1, c0 + 1, out_v1, semw1)
                return carry2

            lax.fori_loop(0, n_chunks // 2, pair_body, 0)
            wait_out(out_v0, semw0)
            wait_out(out_v1, semw1)

    return sc_gather


def kernel(Xs, As, attn_kernel):
    B, N, F = Xs.shape
    K = N // 2
    BLK = 512
    keep_values, keep_indices = pl.pallas_call(
        _fused_body,
        grid=(B, N // BLK),
        in_specs=[
            pl.BlockSpec((1, BLK, N), lambda b, i: (b, i, 0)),
            pl.BlockSpec((1, N, F), lambda b, i: (b, 0, 0)),
            pl.BlockSpec((F, 1), lambda b, i: (0, 0)),
        ],
        out_specs=[
            pl.BlockSpec((1, 1, K), lambda b, i: (b, 0, 0)),
            pl.BlockSpec((1, 1, K), lambda b, i: (b, 0, 0)),
        ],
        out_shape=[
            jax.ShapeDtypeStruct((B, 1, K), jnp.float32),
            jax.ShapeDtypeStruct((B, 1, K), jnp.int32),
        ],
        scratch_shapes=[
            pltpu.VMEM((BLK, F), jnp.float32),
            pltpu.VMEM((N, 1), jnp.float32),
            pltpu.VMEM((N, 1), jnp.int32),
        ],
    )(As, Xs, attn_kernel)
    keep_values = jnp.reshape(keep_values, (B, K))
    keep_indices = jnp.reshape(keep_indices, (B, K))

    FP = 128
    Xs_pad = jnp.pad(Xs, ((0, 0), (0, 0), (0, FP - F)))
    As_out, Xs_out_pad = _make_sc_gather(B, N, FP, K)(keep_indices, As, Xs_pad)
    Xs_out = Xs_out_pad[:, :, :F]
    return (Xs_out, As_out, keep_values)
